# initial kernel scaffold (unmeasured)
import jax
import jax.numpy as jnp
from jax import lax
from jax.experimental import pallas as pl
from jax.experimental.pallas import tpu as pltpu

N_DEV = 4


def kernel(x, router_W, route_idx, expert_W, shared_W):
    n, d = x.shape
    n_experts = router_W.shape[1]
    n_local, _, h = expert_W.shape
    chunk = n // N_DEV

    def body(x_ref, rw_ref, idx_ref, ew_ref, sw_ref, out_ref,
             send_buf, recv_buf, wbf_ref, gate_ref, send_sems, recv_sems):
        my = lax.axis_index("i")
        left = lax.rem(my + N_DEV - 1, N_DEV)
        right = lax.rem(my + 1, N_DEV)

        barrier_sem = pltpu.get_barrier_semaphore()
        for nbr in (left, right):
            pl.semaphore_signal(barrier_sem, inc=1, device_id=(nbr,),
                                device_id_type=pl.DeviceIdType.MESH)
        pl.semaphore_wait(barrier_sem, 2)

        xbf = x_ref[...].astype(jnp.bfloat16)
        scores = jnp.dot(xbf, rw_ref[...].astype(jnp.bfloat16),
                         preferred_element_type=jnp.float32)
        p = jnp.exp(scores - jnp.max(scores, axis=-1, keepdims=True))
        p = p / jnp.sum(p, axis=-1, keepdims=True)
        eids = lax.broadcasted_iota(jnp.int32, (n, n_experts), 1)
        gate_ref[...] = jnp.sum(jnp.where(eids == idx_ref[...], p, 0.0),
                                axis=-1, keepdims=True)

        wbf_ref[...] = ew_ref[...].astype(jnp.bfloat16)

        def partial_chunk(c):
            rows = pl.ds(c * chunk, chunk)
            xc = x_ref[rows, :]
            gc = gate_ref[rows, :]
            ic = idx_ref[rows, :]
            acc = jnp.zeros((chunk, h), jnp.float32)
            for e_l in range(n_local):
                w = jnp.where(ic == my * n_local + e_l, gc, 0.0)
                xw = (xc * w).astype(jnp.bfloat16)
                acc = acc + jnp.dot(xw, wbf_ref[e_l],
                                    preferred_element_type=jnp.float32)
            return acc

        send_buf[0, :, :] = partial_chunk(
            lax.rem(my + N_DEV - 1, N_DEV)).astype(jnp.bfloat16)
        rdmas = []
        for t in range(N_DEV - 1):
            rdma = pltpu.make_async_remote_copy(
                src_ref=send_buf.at[t],
                dst_ref=recv_buf.at[t],
                send_sem=send_sems.at[t],
                recv_sem=recv_sems.at[t],
                device_id=(right,),
                device_id_type=pl.DeviceIdType.MESH,
            )
            rdma.start()
            rdmas.append(rdma)
            if t < N_DEV - 2:
                nxt = partial_chunk(lax.rem(my + N_DEV - 2 - t, N_DEV))
                rdma.wait_recv()
                send_buf[t + 1, :, :] = (
                    recv_buf[t, :, :].astype(jnp.float32) + nxt
                ).astype(jnp.bfloat16)
            else:
                own = partial_chunk(my)
                xc_my = x_ref[pl.ds(my * chunk, chunk), :].astype(jnp.bfloat16)
                own = own + jnp.dot(xc_my, sw_ref[...].astype(jnp.bfloat16),
                                    preferred_element_type=jnp.float32)
                rdma.wait_recv()
                out_ref[...] = recv_buf[t, :, :].astype(jnp.float32) + own
        for r in rdmas:
            r.wait_send()

    return pl.pallas_call(
        body,
        out_shape=jax.ShapeDtypeStruct((chunk, h), jnp.float32),
        in_specs=[pl.BlockSpec(memory_space=pltpu.VMEM)] * 5,
        out_specs=pl.BlockSpec(memory_space=pltpu.VMEM),
        scratch_shapes=[
            pltpu.VMEM((N_DEV - 1, chunk, h), jnp.bfloat16),
            pltpu.VMEM((N_DEV - 1, chunk, h), jnp.bfloat16),
            pltpu.VMEM((n_local, d, h), jnp.bfloat16),
            pltpu.VMEM((n, 1), jnp.float32),
            pltpu.SemaphoreType.DMA((N_DEV - 1,)),
            pltpu.SemaphoreType.DMA((N_DEV - 1,)),
        ],
        compiler_params=pltpu.CompilerParams(collective_id=0),
    )(x, router_W, route_idx, expert_W, shared_W)


# baseline (device time: 66028 ns/iter reference)
import jax
import jax.numpy as jnp
from jax import lax
from jax.experimental import pallas as pl
from jax.experimental.pallas import tpu as pltpu

N_DEV = 4


def kernel(x, router_W, route_idx, expert_W, shared_W):
    n, d = x.shape
    n_experts = router_W.shape[1]
    n_local, _, h = expert_W.shape
    chunk = n // N_DEV

    x = x.astype(jnp.bfloat16)
    router_W = router_W.astype(jnp.bfloat16)
    expert_W = expert_W.astype(jnp.bfloat16)
    shared_W = shared_W.astype(jnp.bfloat16)

    def body(x_ref, rw_ref, idx_ref, ew_ref, sw_ref, out_ref,
             send_buf, recv_buf, gate_ref, send_sems, recv_sems):
        my = lax.axis_index("i")
        left = lax.rem(my + N_DEV - 1, N_DEV)
        right = lax.rem(my + 1, N_DEV)

        barrier_sem = pltpu.get_barrier_semaphore()
        for nbr in (left, right):
            pl.semaphore_signal(barrier_sem, inc=1, device_id=(nbr,),
                                device_id_type=pl.DeviceIdType.MESH)
        pl.semaphore_wait(barrier_sem, 2)

        scores = jnp.dot(x_ref[...], rw_ref[...],
                         preferred_element_type=jnp.float32)
        p = jnp.exp(scores - jnp.max(scores, axis=-1, keepdims=True))
        p = p / jnp.sum(p, axis=-1, keepdims=True)
        eids = lax.broadcasted_iota(jnp.int32, (n, n_experts), 1)
        gate_ref[...] = jnp.sum(jnp.where(eids == idx_ref[...], p, 0.0),
                                axis=-1, keepdims=True)

        def partial_chunk(c):
            rows = pl.ds(c * chunk, chunk)
            xc = x_ref[rows, :]
            gc = gate_ref[rows, :].astype(jnp.bfloat16)
            ic = idx_ref[rows, :]
            acc = jnp.zeros((chunk, h), jnp.float32)
            for e_l in range(n_local):
                w = jnp.where(ic == my * n_local + e_l, gc,
                              jnp.bfloat16(0.0))
                acc = acc + jnp.dot(xc * w, ew_ref[e_l],
                                    preferred_element_type=jnp.float32)
            return acc

        send_buf[0, :, :] = partial_chunk(
            lax.rem(my + N_DEV - 1, N_DEV)).astype(jnp.bfloat16)
        rdmas = []
        for t in range(N_DEV - 1):
            rdma = pltpu.make_async_remote_copy(
                src_ref=send_buf.at[t],
                dst_ref=recv_buf.at[t],
                send_sem=send_sems.at[t],
                recv_sem=recv_sems.at[t],
                device_id=(right,),
                device_id_type=pl.DeviceIdType.MESH,
            )
            rdma.start()
            rdmas.append(rdma)
            if t < N_DEV - 2:
                nxt = partial_chunk(lax.rem(my + N_DEV - 2 - t, N_DEV))
                rdma.wait_recv()
                send_buf[t + 1, :, :] = (
                    recv_buf[t, :, :].astype(jnp.float32) + nxt
                ).astype(jnp.bfloat16)
            else:
                own = partial_chunk(my)
                xc_my = x_ref[pl.ds(my * chunk, chunk), :]
                own = own + jnp.dot(xc_my, sw_ref[...],
                                    preferred_element_type=jnp.float32)
                rdma.wait_recv()
                out_ref[...] = recv_buf[t, :, :].astype(jnp.float32) + own
        for r in rdmas:
            r.wait_send()

    return pl.pallas_call(
        body,
        out_shape=jax.ShapeDtypeStruct((chunk, h), jnp.float32),
        in_specs=[pl.BlockSpec(memory_space=pltpu.VMEM)] * 5,
        out_specs=pl.BlockSpec(memory_space=pltpu.VMEM),
        scratch_shapes=[
            pltpu.VMEM((N_DEV - 1, chunk, h), jnp.bfloat16),
            pltpu.VMEM((N_DEV - 1, chunk, h), jnp.bfloat16),
            pltpu.VMEM((n, 1), jnp.float32),
            pltpu.SemaphoreType.DMA((N_DEV - 1,)),
            pltpu.SemaphoreType.DMA((N_DEV - 1,)),
        ],
        compiler_params=pltpu.CompilerParams(collective_id=0),
    )(x, router_W, route_idx, expert_W, shared_W)


# device time: 65227 ns/iter; 1.0123x vs baseline; 1.0123x over previous
import jax
import jax.numpy as jnp
from jax import lax
from jax.experimental import pallas as pl
from jax.experimental.pallas import tpu as pltpu

N_DEV = 4


def kernel(x, router_W, route_idx, expert_W, shared_W):
    n, d = x.shape
    n_experts = router_W.shape[1]
    n_local, _, h = expert_W.shape
    chunk = n // N_DEV

    x = x.astype(jnp.bfloat16)
    router_W = router_W.astype(jnp.bfloat16)
    shared_W = shared_W.astype(jnp.bfloat16)

    def body(x_ref, rw_ref, idx_ref, ew_ref, sw_ref, out_ref,
             send_buf, recv_buf, gate_ref, send_sems, recv_sems):
        my = lax.axis_index("i")
        left = lax.rem(my + N_DEV - 1, N_DEV)
        right = lax.rem(my + 1, N_DEV)

        barrier_sem = pltpu.get_barrier_semaphore()
        for nbr in (left, right):
            pl.semaphore_signal(barrier_sem, inc=1, device_id=(nbr,),
                                device_id_type=pl.DeviceIdType.MESH)
        pl.semaphore_wait(barrier_sem, 2)

        scores = jnp.dot(x_ref[...], rw_ref[...],
                         preferred_element_type=jnp.float32)
        p = jnp.exp(scores - jnp.max(scores, axis=-1, keepdims=True))
        p = p / jnp.sum(p, axis=-1, keepdims=True)
        eids = lax.broadcasted_iota(jnp.int32, (n, n_experts), 1)
        gate_ref[...] = jnp.sum(jnp.where(eids == idx_ref[...], p, 0.0),
                                axis=-1, keepdims=True)

        def partial_chunk(c):
            rows = pl.ds(c * chunk, chunk)
            xc = x_ref[rows, :]
            gc = gate_ref[rows, :].astype(jnp.bfloat16)
            ic = idx_ref[rows, :]
            acc = jnp.zeros((chunk, h), jnp.float32)
            for e_l in range(n_local):
                w = jnp.where(ic == my * n_local + e_l, gc,
                              jnp.bfloat16(0.0))
                acc = acc + jnp.dot(xc * w, ew_ref[e_l].astype(jnp.bfloat16),
                                    preferred_element_type=jnp.float32)
            return acc

        send_buf[0, :, :] = partial_chunk(
            lax.rem(my + N_DEV - 1, N_DEV)).astype(jnp.bfloat16)
        rdmas = []
        for t in range(N_DEV - 1):
            rdma = pltpu.make_async_remote_copy(
                src_ref=send_buf.at[t],
                dst_ref=recv_buf.at[t],
                send_sem=send_sems.at[t],
                recv_sem=recv_sems.at[t],
                device_id=(right,),
                device_id_type=pl.DeviceIdType.MESH,
            )
            rdma.start()
            rdmas.append(rdma)
            if t < N_DEV - 2:
                nxt = partial_chunk(lax.rem(my + N_DEV - 2 - t, N_DEV))
                rdma.wait_recv()
                send_buf[t + 1, :, :] = (
                    recv_buf[t, :, :].astype(jnp.float32) + nxt
                ).astype(jnp.bfloat16)
            else:
                own = partial_chunk(my)
                xc_my = x_ref[pl.ds(my * chunk, chunk), :]
                own = own + jnp.dot(xc_my, sw_ref[...],
                                    preferred_element_type=jnp.float32)
                rdma.wait_recv()
                out_ref[...] = recv_buf[t, :, :].astype(jnp.float32) + own
        for r in rdmas:
            r.wait_send()

    return pl.pallas_call(
        body,
        out_shape=jax.ShapeDtypeStruct((chunk, h), jnp.float32),
        in_specs=[pl.BlockSpec(memory_space=pltpu.VMEM)] * 5,
        out_specs=pl.BlockSpec(memory_space=pltpu.VMEM),
        scratch_shapes=[
            pltpu.VMEM((N_DEV - 1, chunk, h), jnp.bfloat16),
            pltpu.VMEM((N_DEV - 1, chunk, h), jnp.bfloat16),
            pltpu.VMEM((n, 1), jnp.float32),
            pltpu.SemaphoreType.DMA((N_DEV - 1,)),
            pltpu.SemaphoreType.DMA((N_DEV - 1,)),
        ],
        compiler_params=pltpu.CompilerParams(collective_id=0),
    )(x, router_W, route_idx, expert_W, shared_W)


# device time: 54137 ns/iter; 1.2196x vs baseline; 1.2049x over previous
import jax
import jax.numpy as jnp
from jax import lax
from jax.experimental import pallas as pl
from jax.experimental.pallas import tpu as pltpu

N_DEV = 4


def kernel(x, router_W, route_idx, expert_W, shared_W):
    n, d = x.shape
    n_experts = router_W.shape[1]
    n_local, _, h = expert_W.shape
    chunk = n // N_DEV
    hh = h // 2

    x = x.astype(jnp.bfloat16)
    router_W = router_W.astype(jnp.bfloat16)
    shared_W = shared_W.astype(jnp.bfloat16)

    def body(x_ref, rw_ref, idx_ref, ew_ref, sw_ref, out_ref,
             sbr, rbr, sbl, rbl, gate_ref,
             ss_r, rs_r, ss_l, rs_l):
        my = lax.axis_index("i")
        left = lax.rem(my + N_DEV - 1, N_DEV)
        right = lax.rem(my + 1, N_DEV)

        barrier_sem = pltpu.get_barrier_semaphore()
        for nbr in (left, right):
            pl.semaphore_signal(barrier_sem, inc=1, device_id=(nbr,),
                                device_id_type=pl.DeviceIdType.MESH)
        pl.semaphore_wait(barrier_sem, 2)

        scores = jnp.dot(x_ref[...], rw_ref[...],
                         preferred_element_type=jnp.float32)
        p = jnp.exp(scores - jnp.max(scores, axis=-1, keepdims=True))
        p = p / jnp.sum(p, axis=-1, keepdims=True)
        eids = lax.broadcasted_iota(jnp.int32, (n, n_experts), 1)
        gate_ref[...] = jnp.sum(jnp.where(eids == idx_ref[...], p, 0.0),
                                axis=-1, keepdims=True).astype(jnp.bfloat16)

        def partial_chunk(c, c0, cw):
            rows = pl.ds(c * chunk, chunk)
            xc = x_ref[rows, :]
            gc = gate_ref[rows, :]
            ic = idx_ref[rows, :]
            acc = jnp.zeros((chunk, cw), jnp.float32)
            for e_l in range(n_local):
                w = jnp.where(ic == my * n_local + e_l, gc, jnp.bfloat16(0.0))
                acc = acc + jnp.dot(
                    xc * w, ew_ref[e_l, :, c0:c0 + cw].astype(jnp.bfloat16),
                    preferred_element_type=jnp.float32)
            return acc

        def hop(t):
            r = pltpu.make_async_remote_copy(
                src_ref=sbr.at[t], dst_ref=rbr.at[t],
                send_sem=ss_r.at[t], recv_sem=rs_r.at[t],
                device_id=(right,), device_id_type=pl.DeviceIdType.MESH)
            l = pltpu.make_async_remote_copy(
                src_ref=sbl.at[t], dst_ref=rbl.at[t],
                send_sem=ss_l.at[t], recv_sem=rs_l.at[t],
                device_id=(left,), device_id_type=pl.DeviceIdType.MESH)
            r.start()
            l.start()
            return r, l

        sbr[0, :, :] = partial_chunk(lax.rem(my + 3, N_DEV), 0, hh
                                     ).astype(jnp.bfloat16)
        sbl[0, :, :] = partial_chunk(lax.rem(my + 1, N_DEV), hh, hh
                                     ).astype(jnp.bfloat16)

        r0, l0 = hop(0)
        acc2 = partial_chunk(lax.rem(my + 2, N_DEV), 0, h)
        r0.wait_recv()
        sbr[1, :, :] = (rbr[0, :, :].astype(jnp.float32)
                        + acc2[:, :hh]).astype(jnp.bfloat16)
        l0.wait_recv()
        sbl[1, :, :] = (rbl[0, :, :].astype(jnp.float32)
                        + acc2[:, hh:]).astype(jnp.bfloat16)

        r1, l1 = hop(1)
        a1 = partial_chunk(lax.rem(my + 1, N_DEV), 0, hh)
        b1 = partial_chunk(lax.rem(my + 3, N_DEV), hh, hh)
        r1.wait_recv()
        sbr[2, :, :] = (rbr[1, :, :].astype(jnp.float32) + a1
                        ).astype(jnp.bfloat16)
        l1.wait_recv()
        sbl[2, :, :] = (rbl[1, :, :].astype(jnp.float32) + b1
                        ).astype(jnp.bfloat16)

        r2, l2 = hop(2)
        own = partial_chunk(my, 0, h)
        xc_my = x_ref[pl.ds(my * chunk, chunk), :]
        own = own + jnp.dot(xc_my, sw_ref[...],
                            preferred_element_type=jnp.float32)
        r2.wait_recv()
        out_ref[:, :hh] = rbr[2, :, :].astype(jnp.float32) + own[:, :hh]
        l2.wait_recv()
        out_ref[:, hh:] = rbl[2, :, :].astype(jnp.float32) + own[:, hh:]

        for dsc in (r0, l0, r1, l1, r2, l2):
            dsc.wait_send()

    return pl.pallas_call(
        body,
        out_shape=jax.ShapeDtypeStruct((chunk, h), jnp.float32),
        in_specs=[pl.BlockSpec(memory_space=pltpu.VMEM)] * 5,
        out_specs=pl.BlockSpec(memory_space=pltpu.VMEM),
        scratch_shapes=[
            pltpu.VMEM((N_DEV - 1, chunk, hh), jnp.bfloat16),
            pltpu.VMEM((N_DEV - 1, chunk, hh), jnp.bfloat16),
            pltpu.VMEM((N_DEV - 1, chunk, hh), jnp.bfloat16),
            pltpu.VMEM((N_DEV - 1, chunk, hh), jnp.bfloat16),
            pltpu.VMEM((n, 1), jnp.bfloat16),
            pltpu.SemaphoreType.DMA((N_DEV - 1,)),
            pltpu.SemaphoreType.DMA((N_DEV - 1,)),
            pltpu.SemaphoreType.DMA((N_DEV - 1,)),
            pltpu.SemaphoreType.DMA((N_DEV - 1,)),
        ],
        compiler_params=pltpu.CompilerParams(collective_id=0),
    )(x, router_W, route_idx, expert_W, shared_W)
